# baseline (speedup 1.0000x reference)
import functools

import jax
import jax.numpy as jnp
from jax import lax
from jax.experimental import pallas as pl
from jax.experimental.pallas import tpu as pltpu
from jax.experimental.pallas import tpu_sc as plsc

_B = 16384
_E = 64
_NW = 32
_BPW = _B // _NW
_CH = 128
_NCH = _BPW // _CH
_BM = 2048


def _sc_gather(u2d, i2d, t_mfu, t_mfi, t_mlpu, t_mlpi):
    f32 = jnp.float32
    mesh = plsc.VectorSubcoreMesh(core_axis_name="c", subcore_axis_name="s")

    @functools.partial(
        pl.kernel,
        mesh=mesh,
        compiler_params=pltpu.CompilerParams(use_tc_tiling_on_sc=False),
        out_type=[jax.ShapeDtypeStruct((_B, _E), f32),
                  jax.ShapeDtypeStruct((_B, _E), f32),
                  jax.ShapeDtypeStruct((_B, _E), jnp.bfloat16),
                  jax.ShapeDtypeStruct((_B, _E), jnp.bfloat16)],
        scratch_types=[
            pltpu.VMEM((_NCH, _CH), jnp.int32),
            pltpu.VMEM((_NCH, _CH), jnp.int32),
            pltpu.VMEM((_BPW, _E), f32),
            pltpu.VMEM((_BPW, _E), f32),
            pltpu.VMEM((_BPW, _E), jnp.bfloat16),
            pltpu.VMEM((_BPW, _E), jnp.bfloat16),
            pltpu.SemaphoreType.DMA,
            pltpu.SemaphoreType.DMA,
        ],
    )
    def k(u_hbm, i_hbm, mfu_hbm, mfi_hbm, mlpu_hbm, mlpi_hbm,
          o_mfu, o_mfi, o_mlpu, o_mlpi, uv, iv, buf0, buf1, bbuf0, bbuf1,
          s0, s1):
        wid = lax.axis_index("s") * 2 + lax.axis_index("c")
        base = wid * _BPW
        pltpu.sync_copy(u_hbm.at[pl.ds(wid * _NCH, _NCH)], uv)
        pltpu.sync_copy(i_hbm.at[pl.ds(wid * _NCH, _NCH)], iv)

        def fire(tbl, idxv, buf, sem):
            return [
                pltpu.async_copy(tbl.at[idxv.at[c]],
                                 buf.at[pl.ds(c * _CH, _CH)], sem)
                for c in range(_NCH)
            ]

        def drain(handles):
            for h in handles:
                h.wait()

        h0 = fire(mfu_hbm, uv, buf0, s0)
        h1 = fire(mlpu_hbm, uv, bbuf1, s1)
        drain(h0)
        pltpu.sync_copy(buf0, o_mfu.at[pl.ds(base, _BPW)])
        h0 = fire(mfi_hbm, iv, buf0, s0)
        drain(h1)
        pltpu.sync_copy(bbuf1, o_mlpu.at[pl.ds(base, _BPW)])
        h1 = fire(mlpi_hbm, iv, bbuf0, s1)
        drain(h0)
        pltpu.sync_copy(buf0, o_mfi.at[pl.ds(base, _BPW)])
        drain(h1)
        pltpu.sync_copy(bbuf0, o_mlpi.at[pl.ds(base, _BPW)])

    return k(u2d, i2d, t_mfu, t_mfi, t_mlpu, t_mlpi)


def _mlp_body(mfu_ref, mfi_ref, mlpu_ref, mlpi_ref, w0_ref, b0_ref,
              w1_ref, b1_ref, w2_ref, b2_ref, w3_ref, b3_ref,
              wn_ref, bn_ref, out_ref):
    f32 = jnp.float32
    mlpu = mlpu_ref[...].astype(f32)
    mlpi = mlpi_ref[...].astype(f32)
    w0 = w0_ref[...]
    h = (jnp.dot(mlpu, w0[:_E], preferred_element_type=f32)
         + jnp.dot(mlpi, w0[_E:], preferred_element_type=f32)
         + b0_ref[...])
    h = jnp.maximum(h, 0.0)
    h = jnp.maximum(
        jnp.dot(h, w1_ref[...], preferred_element_type=f32) + b1_ref[...], 0.0)
    h = jnp.maximum(
        jnp.dot(h, w2_ref[...], preferred_element_type=f32) + b2_ref[...], 0.0)
    h = jnp.dot(h, w3_ref[...], preferred_element_type=f32) + b3_ref[...]
    gmf = mfu_ref[...] * mfi_ref[...]
    wn = wn_ref[...]
    logit = (jnp.sum(gmf * wn[:, :_E], axis=1)
             + jnp.sum(h * wn[:, _E:], axis=1) + bn_ref[0, 0])
    out_ref[...] = 1.0 / (1.0 + jnp.exp(-logit))


def _tc_mlp(mfu, mfi, mlpu, mlpi, W0, b0, W1, b1, W2, b2, W3, b3, Wn, bn):
    grid = (_B // _BM,)
    batch_spec = pl.BlockSpec((_BM, _E), lambda i: (i, 0))

    def full(a):
        return pl.BlockSpec(a.shape, lambda i: tuple(0 for _ in a.shape))

    in_specs = [batch_spec] * 4 + [
        full(W0), full(b0), full(W1), full(b1), full(W2), full(b2),
        full(W3), full(b3), full(Wn), full(bn),
    ]
    return pl.pallas_call(
        _mlp_body,
        grid=grid,
        in_specs=in_specs,
        out_specs=pl.BlockSpec((_BM,), lambda i: (i,)),
        out_shape=jax.ShapeDtypeStruct((_B,), jnp.float32),
    )(mfu, mfi, mlpu, mlpi, W0, b0, W1, b1, W2, b2, W3, b3, Wn, bn)


def kernel(user_indices, item_indices, mf_user_table, mf_item_table,
           mlp_user_table, mlp_item_table, W0, b0, W1, b1, W2, b2, W3, b3,
           Wn, bn):
    u2d = user_indices.astype(jnp.int32).reshape(_B // _CH, _CH)
    i2d = item_indices.astype(jnp.int32).reshape(_B // _CH, _CH)
    mfu, mfi, mlpu, mlpi = _sc_gather(
        u2d, i2d, mf_user_table, mf_item_table,
        mlp_user_table.astype(jnp.bfloat16),
        mlp_item_table.astype(jnp.bfloat16))
    return _tc_mlp(
        mfu, mfi, mlpu, mlpi, W0, b0.reshape(1, -1), W1, b1.reshape(1, -1),
        W2, b2.reshape(1, -1), W3, b3.reshape(1, -1), Wn.reshape(1, -1),
        bn.reshape(1, 1))


# speedup vs baseline: 1.4588x; 1.4588x over previous
import functools

import jax
import jax.numpy as jnp
from jax import lax
from jax.experimental import pallas as pl
from jax.experimental.pallas import tpu as pltpu
from jax.experimental.pallas import tpu_sc as plsc

_B = 16384
_E = 64
_NW = 32
_BPW = _B // _NW
_BM = 2048


def _sc_gather_one(idx, table):
    mesh = plsc.VectorSubcoreMesh(core_axis_name="c", subcore_axis_name="s")

    @functools.partial(
        pl.kernel,
        mesh=mesh,
        out_type=jax.ShapeDtypeStruct((_B, _E), jnp.float32),
        scratch_types=[
            pltpu.VMEM((_BPW,), jnp.int32),
            pltpu.SemaphoreType.DMA,
        ],
    )
    def k(i_hbm, tbl_hbm, out, iv, sem):
        wid = lax.axis_index("s") * 2 + lax.axis_index("c")
        base = wid * _BPW
        pltpu.sync_copy(i_hbm.at[pl.ds(base, _BPW)], iv)

        def body(c, carry):
            v = iv[pl.ds(c * 16, 16)]
            for j in range(16):
                r = v[j]
                pltpu.async_copy(
                    tbl_hbm.at[pl.ds(r, 1)],
                    out.at[pl.ds(base + c * 16 + j, 1)],
                    sem)
            return carry

        lax.fori_loop(0, _BPW // 16, body, 0)
        pltpu.make_async_copy(
            tbl_hbm.at[pl.ds(0, _BPW)],
            out.at[pl.ds(base, _BPW)], sem).wait()

    return k(idx, table)


def _mlp_body(mfu_ref, mfi_ref, mlpu_ref, mlpi_ref, w0_ref, b0_ref,
              w1_ref, b1_ref, w2_ref, b2_ref, w3_ref, b3_ref,
              wn_ref, bn_ref, out_ref):
    f32 = jnp.float32
    mlpu = mlpu_ref[...]
    mlpi = mlpi_ref[...]
    w0 = w0_ref[...]
    h = (jnp.dot(mlpu, w0[:_E], preferred_element_type=f32)
         + jnp.dot(mlpi, w0[_E:], preferred_element_type=f32)
         + b0_ref[...])
    h = jnp.maximum(h, 0.0)
    h = jnp.maximum(
        jnp.dot(h, w1_ref[...], preferred_element_type=f32) + b1_ref[...], 0.0)
    h = jnp.maximum(
        jnp.dot(h, w2_ref[...], preferred_element_type=f32) + b2_ref[...], 0.0)
    h = jnp.dot(h, w3_ref[...], preferred_element_type=f32) + b3_ref[...]
    gmf = mfu_ref[...] * mfi_ref[...]
    wn = wn_ref[...]
    logit = (jnp.sum(gmf * wn[:, :_E], axis=1)
             + jnp.sum(h * wn[:, _E:], axis=1) + bn_ref[0, 0])
    out_ref[...] = 1.0 / (1.0 + jnp.exp(-logit))


def _tc_mlp(mfu, mfi, mlpu, mlpi, W0, b0, W1, b1, W2, b2, W3, b3, Wn, bn):
    grid = (_B // _BM,)
    batch_spec = pl.BlockSpec((_BM, _E), lambda i: (i, 0))

    def full(a):
        return pl.BlockSpec(a.shape, lambda i: tuple(0 for _ in a.shape))

    in_specs = [batch_spec] * 4 + [
        full(W0), full(b0), full(W1), full(b1), full(W2), full(b2),
        full(W3), full(b3), full(Wn), full(bn),
    ]
    return pl.pallas_call(
        _mlp_body,
        grid=grid,
        in_specs=in_specs,
        out_specs=pl.BlockSpec((_BM,), lambda i: (i,)),
        out_shape=jax.ShapeDtypeStruct((_B,), jnp.float32),
    )(mfu, mfi, mlpu, mlpi, W0, b0, W1, b1, W2, b2, W3, b3, Wn, bn)


def kernel(user_indices, item_indices, mf_user_table, mf_item_table,
           mlp_user_table, mlp_item_table, W0, b0, W1, b1, W2, b2, W3, b3,
           Wn, bn):
    uidx = user_indices.astype(jnp.int32)
    iidx = item_indices.astype(jnp.int32)
    mfu = _sc_gather_one(uidx, mf_user_table)
    mfi = _sc_gather_one(iidx, mf_item_table)
    mlpu = _sc_gather_one(uidx, mlp_user_table)
    mlpi = _sc_gather_one(iidx, mlp_item_table)
    return _tc_mlp(
        mfu, mfi, mlpu, mlpi, W0, b0.reshape(1, -1), W1, b1.reshape(1, -1),
        W2, b2.reshape(1, -1), W3, b3.reshape(1, -1), Wn.reshape(1, -1),
        bn.reshape(1, 1))


# speedup vs baseline: 1.7011x; 1.1661x over previous
import functools

import jax
import jax.numpy as jnp
from jax import lax
from jax.experimental import pallas as pl
from jax.experimental.pallas import tpu as pltpu
from jax.experimental.pallas import tpu_sc as plsc

_B = 16384
_E = 64
_NW = 32
_BPW = _B // _NW
_BM = 2048


def _sc_gather_one(idx, table):
    mesh = plsc.VectorSubcoreMesh(core_axis_name="c", subcore_axis_name="s")

    @functools.partial(
        pl.kernel,
        mesh=mesh,
        out_type=jax.ShapeDtypeStruct((_B, _E), jnp.float32),
        scratch_types=[
            pltpu.VMEM((_BPW,), jnp.int32),
            pltpu.VMEM((_BPW, _E), jnp.float32),
            pltpu.SemaphoreType.DMA,
        ],
    )
    def k(i_hbm, tbl_hbm, out, iv, buf, sem):
        wid = lax.axis_index("s") * 2 + lax.axis_index("c")
        base = wid * _BPW
        pltpu.sync_copy(i_hbm.at[pl.ds(base, _BPW)], iv)

        def body(c, carry):
            v = iv[pl.ds(c * 16, 16)]
            for j in range(16):
                r = v[j]
                pltpu.async_copy(
                    tbl_hbm.at[pl.ds(r, 1)],
                    buf.at[pl.ds(c * 16 + j, 1)],
                    sem)
            return carry

        lax.fori_loop(0, _BPW // 16, body, 0)
        pltpu.make_async_copy(
            tbl_hbm.at[pl.ds(0, _BPW)], buf, sem).wait()
        pltpu.sync_copy(buf, out.at[pl.ds(base, _BPW)])

    return k(idx, table)


def _mlp_body(mfu_ref, mfi_ref, mlpu_ref, mlpi_ref, w0_ref, b0_ref,
              w1_ref, b1_ref, w2_ref, b2_ref, w3_ref, b3_ref,
              wn_ref, bn_ref, out_ref):
    f32 = jnp.float32
    mlpu = mlpu_ref[...]
    mlpi = mlpi_ref[...]
    w0 = w0_ref[...]
    h = (jnp.dot(mlpu, w0[:_E], preferred_element_type=f32)
         + jnp.dot(mlpi, w0[_E:], preferred_element_type=f32)
         + b0_ref[...])
    h = jnp.maximum(h, 0.0)
    h = jnp.maximum(
        jnp.dot(h, w1_ref[...], preferred_element_type=f32) + b1_ref[...], 0.0)
    h = jnp.maximum(
        jnp.dot(h, w2_ref[...], preferred_element_type=f32) + b2_ref[...], 0.0)
    h = jnp.dot(h, w3_ref[...], preferred_element_type=f32) + b3_ref[...]
    gmf = mfu_ref[...] * mfi_ref[...]
    wn = wn_ref[...]
    logit = (jnp.sum(gmf * wn[:, :_E], axis=1)
             + jnp.sum(h * wn[:, _E:], axis=1) + bn_ref[0, 0])
    out_ref[...] = 1.0 / (1.0 + jnp.exp(-logit))


def _tc_mlp(mfu, mfi, mlpu, mlpi, W0, b0, W1, b1, W2, b2, W3, b3, Wn, bn):
    grid = (_B // _BM,)
    batch_spec = pl.BlockSpec((_BM, _E), lambda i: (i, 0))

    def full(a):
        return pl.BlockSpec(a.shape, lambda i: tuple(0 for _ in a.shape))

    in_specs = [batch_spec] * 4 + [
        full(W0), full(b0), full(W1), full(b1), full(W2), full(b2),
        full(W3), full(b3), full(Wn), full(bn),
    ]
    return pl.pallas_call(
        _mlp_body,
        grid=grid,
        in_specs=in_specs,
        out_specs=pl.BlockSpec((_BM,), lambda i: (i,)),
        out_shape=jax.ShapeDtypeStruct((_B,), jnp.float32),
    )(mfu, mfi, mlpu, mlpi, W0, b0, W1, b1, W2, b2, W3, b3, Wn, bn)


def kernel(user_indices, item_indices, mf_user_table, mf_item_table,
           mlp_user_table, mlp_item_table, W0, b0, W1, b1, W2, b2, W3, b3,
           Wn, bn):
    uidx = user_indices.astype(jnp.int32)
    iidx = item_indices.astype(jnp.int32)
    mfu = _sc_gather_one(uidx, mf_user_table)
    mfi = _sc_gather_one(iidx, mf_item_table)
    mlpu = _sc_gather_one(uidx, mlp_user_table)
    mlpi = _sc_gather_one(iidx, mlp_item_table)
    return _tc_mlp(
        mfu, mfi, mlpu, mlpi, W0, b0.reshape(1, -1), W1, b1.reshape(1, -1),
        W2, b2.reshape(1, -1), W3, b3.reshape(1, -1), Wn.reshape(1, -1),
        bn.reshape(1, 1))
